# Initial kernel scaffold; baseline (speedup 1.0000x reference)
import functools

import jax
import jax.numpy as jnp
from jax import lax
from jax.experimental import pallas as pl
from jax.experimental.pallas import tpu as pltpu
from jax.experimental.pallas import tpu_sc as plsc

M, F, C, B, R = 10000, 3072, 100, 1024, 1024

NC, NS = 2, 16
NW = NC * NS
BPW = B // NW
RPW = R // NW
ROWS_BLK = 400


def _copy_body(img_in, logits_in, label_in, img_out, logits_out, label_out):
    img_out[...] = img_in[...]
    logits_out[...] = logits_in[...]
    label_out[...] = label_in[...]


def _copy3(mem_img, mem_logits, mem_label2d):
    grid = (M // ROWS_BLK,)
    specs = [
        pl.BlockSpec((ROWS_BLK, F), lambda i: (i, 0)),
        pl.BlockSpec((ROWS_BLK, C), lambda i: (i, 0)),
        pl.BlockSpec((ROWS_BLK, 1), lambda i: (i, 0)),
    ]
    return pl.pallas_call(
        _copy_body,
        grid=grid,
        in_specs=specs,
        out_specs=specs,
        out_shape=[
            jax.ShapeDtypeStruct((M, F), jnp.float32),
            jax.ShapeDtypeStruct((M, C), jnp.float32),
            jax.ShapeDtypeStruct((M, 1), jnp.int32),
        ],
    )(mem_img, mem_logits, mem_label2d)


def _sc_body(img_ref, logits_ref, label_ref,
             x_hbm, logits_hbm, y_hbm, idx_hbm, ridx_hbm,
             rx_hbm, rl_hbm, ry_hbm,
             idx_v, y_v, winner_v, wsel_v, tsel_v, ysel_v,
             ri_v, rlab_v, rows_v, lrow_v,
             sem0, sem1, sem2):
    wid = lax.axis_index("s") * NC + lax.axis_index("c")
    base = wid * BPW

    cp_idx = pltpu.async_copy(idx_hbm, idx_v, sem0)
    cp_y = pltpu.async_copy(y_hbm, y_v, sem1)
    cp_ri = pltpu.async_copy(ridx_hbm.at[pl.ds(base, RPW)], ri_v, sem2)

    def _zero(i, _):
        winner_v[pl.ds(i * 16, 16)] = jnp.zeros((16,), jnp.int32)
        return 0
    lax.fori_loop(0, M // 16, _zero, 0)
    cp_idx.wait()
    cp_y.wait()
    cp_ri.wait()

    def _build(b, _):
        winner_v[idx_v[b]] = b + 1
        return 0
    lax.fori_loop(0, B, _build, 0)

    def _sel(j, _):
        t = idx_v[base + j]
        win = winner_v[t] - 1
        wsel_v[j] = win
        tsel_v[j] = t
        ysel_v[j] = y_v[win]
        return 0
    lax.fori_loop(0, BPW, _sel, 0)

    pltpu.async_copy(x_hbm.at[wsel_v], rows_v, sem0).wait()
    pltpu.async_copy(logits_hbm.at[wsel_v], lrow_v, sem1).wait()
    cs0 = pltpu.async_copy(rows_v, img_ref.at[tsel_v], sem0)
    cs1 = pltpu.async_copy(lrow_v, logits_ref.at[tsel_v], sem1)
    cs2 = pltpu.async_copy(ysel_v, label_ref.at[tsel_v], sem2)
    cs0.wait()
    cs1.wait()
    cs2.wait()

    pltpu.async_copy(img_ref.at[ri_v], rows_v, sem0).wait()
    pltpu.async_copy(logits_ref.at[ri_v], lrow_v, sem1).wait()
    pltpu.async_copy(label_ref.at[ri_v], rlab_v, sem2).wait()

    def _patch(j, _):
        r = ri_v[j]
        win = winner_v[r] - 1

        @pl.when(win >= 0)
        def _():
            pltpu.sync_copy(x_hbm.at[pl.ds(win, 1)], rows_v.at[pl.ds(j, 1)])
            pltpu.sync_copy(logits_hbm.at[pl.ds(win, 1)],
                            lrow_v.at[pl.ds(j, 1)])
            rlab_v[j] = y_v[win]
        return 0
    lax.fori_loop(0, RPW, _patch, 0)

    pltpu.sync_copy(rows_v, rx_hbm.at[pl.ds(base, RPW)])
    pltpu.sync_copy(lrow_v, rl_hbm.at[pl.ds(base, RPW)])
    pltpu.sync_copy(rlab_v, ry_hbm.at[pl.ds(base, RPW)])


_sc_call = functools.partial(
    pl.kernel,
    out_type=(
        jax.ShapeDtypeStruct((R, F), jnp.float32),
        jax.ShapeDtypeStruct((R, C), jnp.float32),
        jax.ShapeDtypeStruct((R,), jnp.int32),
    ),
    mesh=plsc.VectorSubcoreMesh(core_axis_name="c", subcore_axis_name="s"),
    scratch_types=[
        pltpu.VMEM((B,), jnp.int32),
        pltpu.VMEM((B,), jnp.int32),
        pltpu.VMEM((M,), jnp.int32),
        pltpu.VMEM((BPW,), jnp.int32),
        pltpu.VMEM((BPW,), jnp.int32),
        pltpu.VMEM((BPW,), jnp.int32),
        pltpu.VMEM((RPW,), jnp.int32),
        pltpu.VMEM((RPW,), jnp.int32),
        pltpu.VMEM((BPW, F), jnp.float32),
        pltpu.VMEM((BPW, C), jnp.float32),
        pltpu.SemaphoreType.DMA,
        pltpu.SemaphoreType.DMA,
        pltpu.SemaphoreType.DMA,
    ],
)(_sc_body)


def kernel(mem_img, mem_logits, mem_label, x, logits, y, idx, retrieve_idx):
    img_c, logits_c, label_c = _copy3(mem_img, mem_logits,
                                      mem_label.reshape(M, 1))
    img_r = jax.new_ref(img_c)
    logits_r = jax.new_ref(logits_c)
    label_r = jax.new_ref(label_c.reshape(M))
    r_x, r_l, r_y = _sc_call(img_r, logits_r, label_r,
                             x, logits, y, idx, retrieve_idx)
    return (jax.freeze(img_r), jax.freeze(logits_r), jax.freeze(label_r),
            r_x, r_l, r_y)


# speedup vs baseline: 2.1890x; 2.1890x over previous
import functools

import jax
import jax.numpy as jnp
from jax import lax
from jax.experimental import pallas as pl
from jax.experimental.pallas import tpu as pltpu
from jax.experimental.pallas import tpu_sc as plsc

M, F, C, B, R = 10000, 3072, 100, 1024, 1024
CP = 128

NC, NS = 2, 16
NW = NC * NS
BPW = B // NW
RPW = R // NW
ROWS_BLK = 400


def _copy_body(img_in, logits_in, label_in, img_out, logits_out, label_out):
    img_out[...] = img_in[...]
    logits_out[...] = logits_in[...]
    label_out[...] = label_in[...]


def _copy3(mem_img, mem_logits, mem_label2d):
    grid = (M // ROWS_BLK,)
    specs = [
        pl.BlockSpec((ROWS_BLK, F), lambda i: (i, 0)),
        pl.BlockSpec((ROWS_BLK, CP), lambda i: (i, 0)),
        pl.BlockSpec((ROWS_BLK, 1), lambda i: (i, 0)),
    ]
    return pl.pallas_call(
        _copy_body,
        grid=grid,
        in_specs=specs,
        out_specs=specs,
        out_shape=[
            jax.ShapeDtypeStruct((M, F), jnp.float32),
            jax.ShapeDtypeStruct((M, CP), jnp.float32),
            jax.ShapeDtypeStruct((M, 1), jnp.int32),
        ],
    )(mem_img, mem_logits, mem_label2d)


def _sc_body(img_ref, logits_ref, label_ref,
             x_hbm, logits_hbm, y_hbm, idx_hbm, ridx_hbm,
             rx_hbm, rl_hbm, ry_hbm,
             idx_v, y_v, winner_v, wsel_v, tsel_v, ysel_v,
             ri_v, rlab_v, pwin_v, rows_v, lrow_v,
             sem0, sem1, sem2):
    wid = lax.axis_index("s") * NC + lax.axis_index("c")
    base = wid * BPW

    cp_idx = pltpu.async_copy(idx_hbm, idx_v.at[pl.ds(0, B)], sem0)
    cp_y = pltpu.async_copy(y_hbm, y_v, sem1)
    cp_ri = pltpu.async_copy(ridx_hbm.at[pl.ds(base, RPW)], ri_v, sem2)

    def _zero(i, _):
        winner_v[pl.ds(i * 16, 16)] = jnp.zeros((16,), jnp.int32)
        return 0
    lax.fori_loop(0, M // 16, _zero, 0)
    cp_idx.wait()
    cp_y.wait()
    cp_ri.wait()

    lane0 = lax.iota(jnp.int32, 16) == 0

    def _build(b, _):
        tvec = idx_v[pl.ds(b, 16)]
        plsc.store_scatter(winner_v, [tvec],
                           jnp.full((16,), 0, jnp.int32) + (b + 1),
                           mask=lane0)
        return 0
    lax.fori_loop(0, B, _build, 0)

    for k in range(BPW // 16):
        tk = idx_v[pl.ds(base + k * 16, 16)]
        wk = plsc.load_gather(winner_v, [tk]) - 1
        tsel_v[pl.ds(k * 16, 16)] = tk
        wsel_v[pl.ds(k * 16, 16)] = wk
        ysel_v[pl.ds(k * 16, 16)] = plsc.load_gather(y_v, [wk])

    pltpu.async_copy(x_hbm.at[wsel_v], rows_v, sem0).wait()
    pltpu.async_copy(logits_hbm.at[wsel_v], lrow_v, sem1).wait()
    cs0 = pltpu.async_copy(rows_v, img_ref.at[tsel_v], sem0)
    cs1 = pltpu.async_copy(lrow_v, logits_ref.at[tsel_v], sem1)
    cs2 = pltpu.async_copy(ysel_v, label_ref.at[tsel_v], sem2)
    cs0.wait()
    cs1.wait()
    cs2.wait()

    pltpu.async_copy(img_ref.at[ri_v], rows_v, sem0).wait()
    pltpu.async_copy(logits_ref.at[ri_v], lrow_v, sem1).wait()
    pltpu.async_copy(label_ref.at[ri_v], rlab_v, sem2).wait()

    for k in range(RPW // 16):
        rk = ri_v[pl.ds(k * 16, 16)]
        wk = plsc.load_gather(winner_v, [rk]) - 1
        pwin_v[pl.ds(k * 16, 16)] = wk
        ylk = plsc.load_gather(y_v, [jnp.maximum(wk, 0)])
        cur = rlab_v[pl.ds(k * 16, 16)]
        rlab_v[pl.ds(k * 16, 16)] = jnp.where(wk >= 0, ylk, cur)

    for k in range(RPW // 16):
        wvec = pwin_v[pl.ds(k * 16, 16)]
        for lane in range(16):
            win = wvec[lane]
            j = k * 16 + lane

            @pl.when(win >= 0)
            def _(win=win, j=j):
                pltpu.sync_copy(x_hbm.at[pl.ds(win, 1)],
                                rows_v.at[pl.ds(j, 1)])
                pltpu.sync_copy(logits_hbm.at[pl.ds(win, 1)],
                                lrow_v.at[pl.ds(j, 1)])

    pltpu.sync_copy(rows_v, rx_hbm.at[pl.ds(base, RPW)])
    pltpu.sync_copy(lrow_v, rl_hbm.at[pl.ds(base, RPW)])
    pltpu.sync_copy(rlab_v, ry_hbm.at[pl.ds(base, RPW)])


_sc_call = functools.partial(
    pl.kernel,
    out_type=(
        jax.ShapeDtypeStruct((R, F), jnp.float32),
        jax.ShapeDtypeStruct((R, CP), jnp.float32),
        jax.ShapeDtypeStruct((R,), jnp.int32),
    ),
    mesh=plsc.VectorSubcoreMesh(core_axis_name="c", subcore_axis_name="s"),
    compiler_params=pltpu.CompilerParams(needs_layout_passes=False),
    scratch_types=[
        pltpu.VMEM((B + 16,), jnp.int32),
        pltpu.VMEM((B,), jnp.int32),
        pltpu.VMEM((M,), jnp.int32),
        pltpu.VMEM((BPW,), jnp.int32),
        pltpu.VMEM((BPW,), jnp.int32),
        pltpu.VMEM((BPW,), jnp.int32),
        pltpu.VMEM((RPW,), jnp.int32),
        pltpu.VMEM((RPW,), jnp.int32),
        pltpu.VMEM((RPW,), jnp.int32),
        pltpu.VMEM((BPW, F), jnp.float32),
        pltpu.VMEM((BPW, CP), jnp.float32),
        pltpu.SemaphoreType.DMA,
        pltpu.SemaphoreType.DMA,
        pltpu.SemaphoreType.DMA,
    ],
)(_sc_body)


def kernel(mem_img, mem_logits, mem_label, x, logits, y, idx, retrieve_idx):
    mem_logits_p = jnp.pad(mem_logits, ((0, 0), (0, CP - C)))
    logits_p = jnp.pad(logits, ((0, 0), (0, CP - C)))
    img_c, logits_c, label_c = _copy3(mem_img, mem_logits_p,
                                      mem_label.reshape(M, 1))
    img_r = jax.new_ref(img_c)
    logits_r = jax.new_ref(logits_c)
    label_r = jax.new_ref(label_c.reshape(M))
    r_x, r_l, r_y = _sc_call(img_r, logits_r, label_r,
                             x, logits_p, y, idx, retrieve_idx)
    return (jax.freeze(img_r), jax.freeze(logits_r)[:, :C],
            jax.freeze(label_r), r_x, r_l[:, :C], r_y)


# speedup vs baseline: 2.1990x; 1.0046x over previous
import functools

import jax
import jax.numpy as jnp
from jax import lax
from jax.experimental import pallas as pl
from jax.experimental.pallas import tpu as pltpu
from jax.experimental.pallas import tpu_sc as plsc

M, F, C, B, R = 10000, 3072, 100, 1024, 1024
CP = 128

NC, NS = 2, 16
NW = NC * NS
BPW = B // NW
RPW = R // NW
ROWS_BLK = 1000


def _copy_body(img_in, logits_in, label_in, img_out, logits_out, label_out):
    img_out[...] = img_in[...]
    logits_out[...] = logits_in[...]
    label_out[...] = label_in[...]


def _copy3(mem_img, mem_logits, mem_label2d):
    grid = (M // ROWS_BLK,)
    specs = [
        pl.BlockSpec((ROWS_BLK, F), lambda i: (i, 0)),
        pl.BlockSpec((ROWS_BLK, CP), lambda i: (i, 0)),
        pl.BlockSpec((ROWS_BLK, 1), lambda i: (i, 0)),
    ]
    return pl.pallas_call(
        _copy_body,
        grid=grid,
        in_specs=specs,
        out_specs=specs,
        out_shape=[
            jax.ShapeDtypeStruct((M, F), jnp.float32),
            jax.ShapeDtypeStruct((M, CP), jnp.float32),
            jax.ShapeDtypeStruct((M, 1), jnp.int32),
        ],
    )(mem_img, mem_logits, mem_label2d)


def _sc_body(img_ref, logits_ref, label_ref,
             x_hbm, logits_hbm, y_hbm, idx_hbm, ridx_hbm,
             rx_hbm, rl_hbm, ry_hbm,
             idx_v, y_v, winner_v, wsel_v, tsel_v, ysel_v,
             ri_v, rlab_v, pwin_v, rows_v, lrow_v,
             sem0, sem1, sem2):
    wid = lax.axis_index("s") * NC + lax.axis_index("c")
    base = wid * BPW

    cp_idx = pltpu.async_copy(idx_hbm, idx_v.at[pl.ds(0, B)], sem0)
    cp_y = pltpu.async_copy(y_hbm, y_v, sem1)
    cp_ri = pltpu.async_copy(ridx_hbm.at[pl.ds(base, RPW)], ri_v, sem2)

    def _zero(i, _):
        winner_v[pl.ds(i * 16, 16)] = jnp.zeros((16,), jnp.int32)
        return 0
    lax.fori_loop(0, M // 16, _zero, 0)
    cp_idx.wait()
    cp_y.wait()
    cp_ri.wait()

    lane0 = lax.iota(jnp.int32, 16) == 0

    def _build(b, _):
        tvec = idx_v[pl.ds(b, 16)]
        plsc.store_scatter(winner_v, [tvec],
                           jnp.full((16,), 0, jnp.int32) + (b + 1),
                           mask=lane0)
        return 0
    lax.fori_loop(0, B, _build, 0)

    for k in range(BPW // 16):
        tk = idx_v[pl.ds(base + k * 16, 16)]
        wk = plsc.load_gather(winner_v, [tk]) - 1
        tsel_v[pl.ds(k * 16, 16)] = tk
        wsel_v[pl.ds(k * 16, 16)] = wk
        ysel_v[pl.ds(k * 16, 16)] = plsc.load_gather(y_v, [wk])

    pltpu.async_copy(x_hbm.at[wsel_v], rows_v, sem0).wait()
    pltpu.async_copy(logits_hbm.at[wsel_v], lrow_v, sem1).wait()
    cs0 = pltpu.async_copy(rows_v, img_ref.at[tsel_v], sem0)
    cs1 = pltpu.async_copy(lrow_v, logits_ref.at[tsel_v], sem1)
    cs2 = pltpu.async_copy(ysel_v, label_ref.at[tsel_v], sem2)
    cs0.wait()
    cs1.wait()
    cs2.wait()

    pltpu.async_copy(img_ref.at[ri_v], rows_v, sem0).wait()
    pltpu.async_copy(logits_ref.at[ri_v], lrow_v, sem1).wait()
    pltpu.async_copy(label_ref.at[ri_v], rlab_v, sem2).wait()

    for k in range(RPW // 16):
        rk = ri_v[pl.ds(k * 16, 16)]
        wk = plsc.load_gather(winner_v, [rk]) - 1
        pwin_v[pl.ds(k * 16, 16)] = wk
        ylk = plsc.load_gather(y_v, [jnp.maximum(wk, 0)])
        cur = rlab_v[pl.ds(k * 16, 16)]
        rlab_v[pl.ds(k * 16, 16)] = jnp.where(wk >= 0, ylk, cur)

    for k in range(RPW // 16):
        wvec = pwin_v[pl.ds(k * 16, 16)]
        for lane in range(16):
            win = wvec[lane]
            j = k * 16 + lane

            @pl.when(win >= 0)
            def _(win=win, j=j):
                pltpu.sync_copy(x_hbm.at[pl.ds(win, 1)],
                                rows_v.at[pl.ds(j, 1)])
                pltpu.sync_copy(logits_hbm.at[pl.ds(win, 1)],
                                lrow_v.at[pl.ds(j, 1)])

    pltpu.sync_copy(rows_v, rx_hbm.at[pl.ds(base, RPW)])
    pltpu.sync_copy(lrow_v, rl_hbm.at[pl.ds(base, RPW)])
    pltpu.sync_copy(rlab_v, ry_hbm.at[pl.ds(base, RPW)])


_sc_call = functools.partial(
    pl.kernel,
    out_type=(
        jax.ShapeDtypeStruct((R, F), jnp.float32),
        jax.ShapeDtypeStruct((R, CP), jnp.float32),
        jax.ShapeDtypeStruct((R,), jnp.int32),
    ),
    mesh=plsc.VectorSubcoreMesh(core_axis_name="c", subcore_axis_name="s"),
    compiler_params=pltpu.CompilerParams(needs_layout_passes=False),
    scratch_types=[
        pltpu.VMEM((B + 16,), jnp.int32),
        pltpu.VMEM((B,), jnp.int32),
        pltpu.VMEM((M,), jnp.int32),
        pltpu.VMEM((BPW,), jnp.int32),
        pltpu.VMEM((BPW,), jnp.int32),
        pltpu.VMEM((BPW,), jnp.int32),
        pltpu.VMEM((RPW,), jnp.int32),
        pltpu.VMEM((RPW,), jnp.int32),
        pltpu.VMEM((RPW,), jnp.int32),
        pltpu.VMEM((BPW, F), jnp.float32),
        pltpu.VMEM((BPW, CP), jnp.float32),
        pltpu.SemaphoreType.DMA,
        pltpu.SemaphoreType.DMA,
        pltpu.SemaphoreType.DMA,
    ],
)(_sc_body)


def kernel(mem_img, mem_logits, mem_label, x, logits, y, idx, retrieve_idx):
    mem_logits_p = jnp.pad(mem_logits, ((0, 0), (0, CP - C)))
    logits_p = jnp.pad(logits, ((0, 0), (0, CP - C)))
    img_c, logits_c, label_c = _copy3(mem_img, mem_logits_p,
                                      mem_label.reshape(M, 1))
    img_r = jax.new_ref(img_c)
    logits_r = jax.new_ref(logits_c)
    label_r = jax.new_ref(label_c.reshape(M))
    r_x, r_l, r_y = _sc_call(img_r, logits_r, label_r,
                             x, logits_p, y, idx, retrieve_idx)
    return (jax.freeze(img_r), jax.freeze(logits_r)[:, :C],
            jax.freeze(label_r), r_x, r_l[:, :C], r_y)
